# Initial kernel scaffold; baseline (speedup 1.0000x reference)
import jax
import jax.numpy as jnp
from jax.experimental import pallas as pl


def kernel(x, edge_index, W1, b1, W2, b2, W3, b3):
    raise NotImplementedError("write your pallas kernel here")


# speedup vs baseline: 17.4494x; 17.4494x over previous
import functools

import jax
import jax.numpy as jnp
from jax import lax
from jax.experimental import pallas as pl
from jax.experimental.pallas import tpu as pltpu
from jax.experimental.pallas import tpu_sc as plsc

N = 10000
E = 320000
D = 128
DW3 = 16
NC = 2
NS = 16
NW = NC * NS
EPW = E // NW
C = 80
NCH = EPW // C
NP = 10240
NPT = NP // NS
BN = 2000


def _scatter_body(g_hbm, src_hbm, dst_hbm, zeros_hbm, out_hbm,
                  srcv, dstv, rows, acc, sem):
    c = lax.axis_index("c")
    s = lax.axis_index("s")
    w = c * NS + s
    pltpu.sync_copy(zeros_hbm, acc.at[pl.ds(s * NPT, NPT)])
    pltpu.sync_copy(src_hbm.at[w], srcv)
    pltpu.sync_copy(dst_hbm.at[w], dstv)
    plsc.subcore_barrier()

    def chunk(i, _):
        pltpu.async_copy(g_hbm.at[srcv.at[i]], rows, sem).wait()
        pltpu.sync_copy(rows, acc.at[dstv.at[i]], add=True)
        return 0

    lax.fori_loop(0, NCH, chunk, 0)
    plsc.subcore_barrier()
    pltpu.sync_copy(acc.at[pl.ds(s * NPT, NPT)],
                    out_hbm.at[pl.ds(c * NP + s * NPT, NPT)])


def _make_scatter(dp):
    mesh = plsc.VectorSubcoreMesh(core_axis_name="c", subcore_axis_name="s")
    return pl.kernel(
        _scatter_body,
        out_type=jax.ShapeDtypeStruct((2 * NP, dp), jnp.float32),
        mesh=mesh,
        scratch_types=[
            pltpu.VMEM((NCH, C), jnp.int32),
            pltpu.VMEM((NCH, C), jnp.int32),
            pltpu.VMEM((C, dp), jnp.float32),
            pltpu.VMEM_SHARED((NP, dp), jnp.float32),
            pltpu.SemaphoreType.DMA,
        ],
        compiler_params=pltpu.CompilerParams(use_tc_tiling_on_sc=False),
    )


_scatter128 = _make_scatter(D)
_scatter16 = _make_scatter(DW3)


def _deg_body(dst_hbm, ones_hbm, zeros_hbm, out_hbm, dstv, onesv, acc):
    c = lax.axis_index("c")
    s = lax.axis_index("s")
    w = c * NS + s
    pltpu.sync_copy(zeros_hbm, acc.at[pl.ds(s * NPT, NPT)])
    pltpu.sync_copy(dst_hbm.at[w], dstv)
    pltpu.sync_copy(ones_hbm, onesv)
    plsc.subcore_barrier()

    def chunk(i, _):
        pltpu.sync_copy(onesv, acc.at[dstv.at[i]], add=True)
        return 0

    lax.fori_loop(0, NCH, chunk, 0)
    plsc.subcore_barrier()
    pltpu.sync_copy(acc.at[pl.ds(s * NPT, NPT)],
                    out_hbm.at[pl.ds(c * NP + s * NPT, NPT)])


_deg_scatter = pl.kernel(
    _deg_body,
    out_type=jax.ShapeDtypeStruct((2 * NP, DW3), jnp.float32),
    mesh=plsc.VectorSubcoreMesh(core_axis_name="c", subcore_axis_name="s"),
    scratch_types=[
        pltpu.VMEM((NCH, C), jnp.int32),
        pltpu.VMEM((C, DW3), jnp.float32),
        pltpu.VMEM_SHARED((NP, DW3), jnp.float32),
    ],
    compiler_params=pltpu.CompilerParams(use_tc_tiling_on_sc=False),
)


def _dinv_body(da_ref, db_ref, t_ref, o_ref):
    deg = da_ref[...] + db_ref[...] + 1.0
    o_ref[...] = jnp.dot(lax.rsqrt(deg), t_ref[...],
                         preferred_element_type=jnp.float32)


def _dinv_kernel(deg_a8, deg_b8, t):
    return pl.pallas_call(
        _dinv_body,
        out_shape=jax.ShapeDtypeStruct((N // 8, 8 * D), jnp.float32),
    )(deg_a8, deg_b8, t)


def _mm1_body(x_ref, w_ref, dv_ref, o_ref):
    o_ref[...] = jnp.dot(x_ref[...], w_ref[...],
                         preferred_element_type=jnp.float32) * dv_ref[...]


def _mm1(x, w1, dinv):
    return pl.pallas_call(
        _mm1_body,
        grid=(N // BN,),
        in_specs=[
            pl.BlockSpec((BN, D), lambda i: (i, 0)),
            pl.BlockSpec((D, D), lambda i: (0, 0)),
            pl.BlockSpec((BN, D), lambda i: (i, 0)),
        ],
        out_specs=pl.BlockSpec((BN, D), lambda i: (i, 0)),
        out_shape=jax.ShapeDtypeStruct((N, D), jnp.float32),
    )(x, w1, dinv)


def _mm_mid_body(aa_ref, ab_ref, g_ref, dv_ref, b_ref, w_ref, dvo_ref, o_ref):
    t = (aa_ref[...] + ab_ref[...] + g_ref[...]) * dv_ref[...] + b_ref[...]
    t = jnp.maximum(t, 0.0)
    o_ref[...] = jnp.dot(t, w_ref[...],
                         preferred_element_type=jnp.float32) * dvo_ref[...]


def _mm_mid(agg_a, agg_b, g, dinv, b, w, dinv_out):
    dw = w.shape[1]
    return pl.pallas_call(
        _mm_mid_body,
        grid=(N // BN,),
        in_specs=[
            pl.BlockSpec((BN, D), lambda i: (i, 0)),
            pl.BlockSpec((BN, D), lambda i: (i, 0)),
            pl.BlockSpec((BN, D), lambda i: (i, 0)),
            pl.BlockSpec((BN, D), lambda i: (i, 0)),
            pl.BlockSpec((1, D), lambda i: (0, 0)),
            pl.BlockSpec((D, dw), lambda i: (0, 0)),
            pl.BlockSpec((BN, dw), lambda i: (i, 0)),
        ],
        out_specs=pl.BlockSpec((BN, dw), lambda i: (i, 0)),
        out_shape=jax.ShapeDtypeStruct((N, dw), jnp.float32),
    )(agg_a, agg_b, g, dinv, b, w, dinv_out)


def _combine_body(aa_ref, ab_ref, g_ref, dv_ref, b_ref, o_ref):
    o_ref[...] = (aa_ref[...] + ab_ref[...] + g_ref[...]) * dv_ref[...] \
        + b_ref[...]


def _combine(agg_a, agg_b, g, dinv16, b3p):
    return pl.pallas_call(
        _combine_body,
        grid=(N // BN,),
        in_specs=[
            pl.BlockSpec((BN, DW3), lambda i: (i, 0)),
            pl.BlockSpec((BN, DW3), lambda i: (i, 0)),
            pl.BlockSpec((BN, DW3), lambda i: (i, 0)),
            pl.BlockSpec((BN, DW3), lambda i: (i, 0)),
            pl.BlockSpec((1, DW3), lambda i: (0, 0)),
        ],
        out_specs=pl.BlockSpec((BN, DW3), lambda i: (i, 0)),
        out_shape=jax.ShapeDtypeStruct((N, DW3), jnp.float32),
    )(agg_a, agg_b, g, dinv16, b3p)


def kernel(x, edge_index, W1, b1, W2, b2, W3, b3):
    src = edge_index[0].reshape(NW, NCH, C)
    dst = edge_index[1].reshape(NW, NCH, C)
    zeros128 = jnp.zeros((NPT, D), jnp.float32)
    zeros16 = jnp.zeros((NPT, DW3), jnp.float32)
    ones16 = jnp.ones((C, DW3), jnp.float32)

    deg2 = _deg_scatter(dst, ones16, zeros16)
    deg_a8 = deg2[:N].reshape(N // 8, 8 * DW3)
    deg_b8 = deg2[NP:NP + N].reshape(N // 8, 8 * DW3)
    col = jnp.arange(8 * D, dtype=jnp.int32) // D
    t = (jnp.arange(D, dtype=jnp.int32)[:, None] == DW3 * col[None, :])
    t = t.astype(jnp.float32)
    dinv = _dinv_kernel(deg_a8, deg_b8, t).reshape(N, D)
    dinv16 = dinv[:, :DW3]

    g1 = _mm1(x, W1, dinv)
    a1 = _scatter128(g1, src, dst, zeros128)
    g2 = _mm_mid(a1[:N], a1[NP:NP + N], g1, dinv, b1.reshape(1, D), W2, dinv)
    a2 = _scatter128(g2, src, dst, zeros128)
    w3p = jnp.zeros((D, DW3), jnp.float32).at[:, :5].set(W3)
    g3 = _mm_mid(a2[:N], a2[NP:NP + N], g2, dinv, b2.reshape(1, D), w3p, dinv16)
    a3 = _scatter16(g3, src, dst, zeros16)
    b3p = jnp.zeros((1, DW3), jnp.float32).at[0, :5].set(b3)
    out16 = _combine(a3[:N], a3[NP:NP + N], g3, dinv16, b3p)
    return out16[:, :5]
